# Initial kernel scaffold; baseline (speedup 1.0000x reference)
import jax
import jax.numpy as jnp
from jax.experimental import pallas as pl


def kernel(obs_encoding, lane_encoding, aggregated_info, same_obs_mask, W1, b1, W2, b2, W3, b3, W4, b4):
    raise NotImplementedError("write your pallas kernel here")


# speedup vs baseline: 3.0769x; 3.0769x over previous
import functools

import jax
import jax.numpy as jnp
from jax import lax
from jax.experimental import pallas as pl
from jax.experimental.pallas import tpu as pltpu
from jax.experimental.pallas import tpu_sc as plsc

N_OBS = 2048
M_LANE = 16384
D_OBS = 128
D_LANE = 128
D_AGG = 1024
H1P = 640
H2P = 128
H3 = 16

LBLK = 1024
SROW = 2048
NROW = M_LANE // SROW
SCHUNK = 512
NEG = -1e30


def _k1_body(obs_ref, agg_ref, w1o_ref, w1a_ref, b1_ref, out_ref):
    out_ref[...] = (
        jnp.dot(obs_ref[...], w1o_ref[...], preferred_element_type=jnp.float32)
        + jnp.dot(agg_ref[...], w1a_ref[...], preferred_element_type=jnp.float32)
        + b1_ref[...]
    )


def _obstacle_head(obs, agg, w1o, w1a, b1p):
    rb = 512
    grid = N_OBS // rb
    return pl.pallas_call(
        _k1_body,
        grid=(grid,),
        in_specs=[
            pl.BlockSpec((rb, D_OBS), lambda i: (i, 0)),
            pl.BlockSpec((rb, D_AGG), lambda i: (i, 0)),
            pl.BlockSpec((D_OBS, H1P), lambda i: (0, 0)),
            pl.BlockSpec((D_AGG, H1P), lambda i: (0, 0)),
            pl.BlockSpec((1, H1P), lambda i: (0, 0)),
        ],
        out_specs=pl.BlockSpec((rb, H1P), lambda i: (i, 0)),
        out_shape=jax.ShapeDtypeStruct((N_OBS, H1P), jnp.float32),
    )(obs, agg, w1o, w1a, b1p)


_NC = 2
_NS = 16
_NW = _NC * _NS
_PER_W = M_LANE // _NW
_GCHUNK = 128


def _sc_gather(table, idx):
    mesh = plsc.VectorSubcoreMesh(core_axis_name="c", subcore_axis_name="s")

    @functools.partial(
        pl.kernel,
        mesh=mesh,
        out_type=jax.ShapeDtypeStruct((M_LANE, H1P), jnp.float32),
        scratch_types=[
            pltpu.VMEM((_GCHUNK,), jnp.int32),
            pltpu.VMEM((_GCHUNK, H1P), jnp.float32),
            pltpu.SemaphoreType.DMA,
        ],
    )
    def gather_kernel(table_hbm, idx_hbm, out_hbm, idx_v, rows_v, sem):
        wid = lax.axis_index("s") * _NC + lax.axis_index("c")
        base = wid * _PER_W
        for c in range(_PER_W // _GCHUNK):
            off = base + c * _GCHUNK
            pltpu.sync_copy(idx_hbm.at[pl.ds(off, _GCHUNK)], idx_v)
            pltpu.async_copy(table_hbm.at[idx_v], rows_v, sem).wait()
            pltpu.sync_copy(rows_v, out_hbm.at[pl.ds(off, _GCHUNK)])

    return gather_kernel(table, idx)


def _k3_body(g_ref, lane_ref, w1l_ref, w2_ref, b2_ref, w3_ref, b3_ref,
             w4_ref, b4_ref, out_ref):
    x1 = g_ref[...] + jnp.dot(lane_ref[...], w1l_ref[...],
                              preferred_element_type=jnp.float32)
    h1 = jnp.maximum(x1, 0.0)
    h2 = jnp.maximum(
        jnp.dot(h1, w2_ref[...], preferred_element_type=jnp.float32)
        + b2_ref[...], 0.0)
    h3 = jnp.maximum(
        jnp.dot(h2, w3_ref[...], preferred_element_type=jnp.float32)
        + b3_ref[...], 0.0)
    out_ref[...] = (jnp.dot(h3, w4_ref[...], preferred_element_type=jnp.float32)
                    + b4_ref[...])


def _mlp_tail(gathered, lane, w1l, w2p, b2p, w3p, b3r, w4, b4r):
    grid = M_LANE // LBLK
    return pl.pallas_call(
        _k3_body,
        grid=(grid,),
        in_specs=[
            pl.BlockSpec((LBLK, H1P), lambda i: (i, 0)),
            pl.BlockSpec((LBLK, D_LANE), lambda i: (i, 0)),
            pl.BlockSpec((D_LANE, H1P), lambda i: (0, 0)),
            pl.BlockSpec((H1P, H2P), lambda i: (0, 0)),
            pl.BlockSpec((1, H2P), lambda i: (0, 0)),
            pl.BlockSpec((H2P, H3), lambda i: (0, 0)),
            pl.BlockSpec((1, H3), lambda i: (0, 0)),
            pl.BlockSpec((H3, 1), lambda i: (0, 0)),
            pl.BlockSpec((1, 1), lambda i: (0, 0)),
        ],
        out_specs=pl.BlockSpec((LBLK, 1), lambda i: (i, 0)),
        out_shape=jax.ShapeDtypeStruct((M_LANE, 1), jnp.float32),
    )(gathered, lane, w1l, w2p, b2p, w3p, b3r, w4, b4r)


def _k4a_body(s_ref, seg_ref, m_ref):
    @pl.when(pl.program_id(0) == 0)
    def _init():
        m_ref[...] = jnp.full((N_OBS, 1), NEG, jnp.float32)

    s_row = s_ref[0]
    seg_row = seg_ref[0]
    for c in range(N_OBS // SCHUNK):
        ids = c * SCHUNK + lax.broadcasted_iota(jnp.int32, (SCHUNK, SROW), 0)
        oh = ids == seg_row
        val = jnp.where(oh, jnp.broadcast_to(s_row, (SCHUNK, SROW)), NEG)
        mx = jnp.max(val, axis=1)[:, None]
        sl = pl.ds(c * SCHUNK, SCHUNK)
        m_ref[sl, :] = jnp.maximum(m_ref[sl, :], mx)


def _k4b_body(s_ref, seg_ref, m_ref, e_ref, den_ref):
    @pl.when(pl.program_id(0) == 0)
    def _init():
        den_ref[...] = jnp.zeros((N_OBS, 1), jnp.float32)

    s_row = s_ref[0]
    seg_row = seg_ref[0]
    m_g = jnp.full((1, SROW), NEG, jnp.float32)
    for c in range(N_OBS // SCHUNK):
        ids = c * SCHUNK + lax.broadcasted_iota(jnp.int32, (SCHUNK, SROW), 0)
        oh = ids == seg_row
        m_c = m_ref[pl.ds(c * SCHUNK, SCHUNK), :]
        g = jnp.where(oh, jnp.broadcast_to(m_c, (SCHUNK, SROW)), NEG)
        m_g = jnp.maximum(m_g, jnp.max(g, axis=0)[None, :])
    e = jnp.exp(s_row - m_g)
    e_ref[0] = e
    for c in range(N_OBS // SCHUNK):
        ids = c * SCHUNK + lax.broadcasted_iota(jnp.int32, (SCHUNK, SROW), 0)
        oh = ids == seg_row
        part = jnp.sum(jnp.where(oh, jnp.broadcast_to(e, (SCHUNK, SROW)), 0.0),
                       axis=1)[:, None]
        sl = pl.ds(c * SCHUNK, SCHUNK)
        den_ref[sl, :] = den_ref[sl, :] + part


def _k4c_body(e_ref, seg_ref, den_ref, out_ref):
    e = e_ref[0]
    seg_row = seg_ref[0]
    d_g = jnp.zeros((1, SROW), jnp.float32)
    for c in range(N_OBS // SCHUNK):
        ids = c * SCHUNK + lax.broadcasted_iota(jnp.int32, (SCHUNK, SROW), 0)
        oh = ids == seg_row
        d_c = den_ref[pl.ds(c * SCHUNK, SCHUNK), :]
        g = jnp.where(oh, jnp.broadcast_to(d_c, (SCHUNK, SROW)), 0.0)
        d_g = jnp.maximum(d_g, jnp.max(g, axis=0)[None, :])
    out_ref[0] = e / d_g


def _segment_softmax(scores, seg):
    s3 = scores.reshape(NROW, 1, SROW)
    seg3 = seg.reshape(NROW, 1, SROW)
    row_spec = pl.BlockSpec((1, 1, SROW), lambda i: (i, 0, 0))
    col_spec = pl.BlockSpec((N_OBS, 1), lambda i: (0, 0))

    seg_max = pl.pallas_call(
        _k4a_body,
        grid=(NROW,),
        in_specs=[row_spec, row_spec],
        out_specs=col_spec,
        out_shape=jax.ShapeDtypeStruct((N_OBS, 1), jnp.float32),
    )(s3, seg3)

    e3, den = pl.pallas_call(
        _k4b_body,
        grid=(NROW,),
        in_specs=[row_spec, row_spec, col_spec],
        out_specs=[row_spec, col_spec],
        out_shape=[
            jax.ShapeDtypeStruct((NROW, 1, SROW), jnp.float32),
            jax.ShapeDtypeStruct((N_OBS, 1), jnp.float32),
        ],
    )(s3, seg3, seg_max)

    out3 = pl.pallas_call(
        _k4c_body,
        grid=(NROW,),
        in_specs=[row_spec, row_spec, col_spec],
        out_specs=row_spec,
        out_shape=jax.ShapeDtypeStruct((NROW, 1, SROW), jnp.float32),
    )(e3, seg3, den)

    return out3.reshape(M_LANE, 1)


def kernel(obs_encoding, lane_encoding, aggregated_info, same_obs_mask,
           W1, b1, W2, b2, W3, b3, W4, b4):
    seg = same_obs_mask[:, 0].astype(jnp.int32)

    w1o = jnp.pad(W1[:D_OBS], ((0, 0), (0, H1P - 600)))
    w1a = jnp.pad(W1[D_OBS:D_OBS + D_AGG], ((0, 0), (0, H1P - 600)))
    w1l = jnp.pad(W1[D_OBS + D_AGG:], ((0, 0), (0, H1P - 600)))
    b1p = jnp.pad(b1, (0, H1P - 600))[None, :]
    w2p = jnp.pad(W2, ((0, H1P - 600), (0, H2P - 100)))
    b2p = jnp.pad(b2, (0, H2P - 100))[None, :]
    w3p = jnp.pad(W3, ((0, H2P - 100), (0, 0)))
    b3r = b3[None, :]
    b4r = b4[None, :]

    h0 = _obstacle_head(obs_encoding, aggregated_info, w1o, w1a, b1p)
    gathered = _sc_gather(h0, seg)
    scores = _mlp_tail(gathered, lane_encoding, w1l, w2p, b2p, w3p, b3r, W4, b4r)
    return _segment_softmax(scores, seg)


# speedup vs baseline: 3.5962x; 1.1688x over previous
import functools

import jax
import jax.numpy as jnp
from jax import lax
from jax.experimental import pallas as pl
from jax.experimental.pallas import tpu as pltpu
from jax.experimental.pallas import tpu_sc as plsc

N_OBS = 2048
M_LANE = 16384
D_OBS = 128
D_LANE = 128
D_AGG = 1024
H1P = 768
H2P = 128
H3 = 16

LBLK = 1024
SROW = 2048
NROW = M_LANE // SROW
SCHUNK = 512
NEG = -1e30

BF = jnp.bfloat16


def _pack_bf16_pair(a, b):
    ba = jax.lax.bitcast_convert_type(a, jnp.int32)
    bb = jax.lax.bitcast_convert_type(b, jnp.int32)
    ba = ba + 32767 + ((ba >> 16) & 1)
    bb = bb + 32767 + ((bb >> 16) & 1)
    return (bb & -65536) | ((ba >> 16) & 65535)


def _unpack_bf16_pair(p):
    a = jax.lax.bitcast_convert_type(p << 16, jnp.float32)
    b = jax.lax.bitcast_convert_type(p & -65536, jnp.float32)
    return a, b


def _k1_body(obs_ref, agg_ref, w1o_ref, w1a_ref, b1_ref, out_ref):
    acc = (
        jnp.dot(obs_ref[...].astype(BF), w1o_ref[...],
                preferred_element_type=jnp.float32)
        + jnp.dot(agg_ref[...].astype(BF), w1a_ref[...],
                  preferred_element_type=jnp.float32)
        + b1_ref[...]
    )
    out_ref[...] = _pack_bf16_pair(acc[:, :H1P // 2], acc[:, H1P // 2:])


def _obstacle_head(obs, agg, w1o, w1a, b1p):
    rb = 512
    grid = N_OBS // rb
    return pl.pallas_call(
        _k1_body,
        grid=(grid,),
        in_specs=[
            pl.BlockSpec((rb, D_OBS), lambda i: (i, 0)),
            pl.BlockSpec((rb, D_AGG), lambda i: (i, 0)),
            pl.BlockSpec((D_OBS, H1P), lambda i: (0, 0)),
            pl.BlockSpec((D_AGG, H1P), lambda i: (0, 0)),
            pl.BlockSpec((1, H1P), lambda i: (0, 0)),
        ],
        out_specs=pl.BlockSpec((rb, H1P // 2), lambda i: (i, 0)),
        out_shape=jax.ShapeDtypeStruct((N_OBS, H1P // 2), jnp.int32),
    )(obs, agg, w1o, w1a, b1p)


_NC = 2
_NS = 16
_NW = _NC * _NS
_PER_W = M_LANE // _NW
_GCHUNK = 128
_NCHUNK = _PER_W // _GCHUNK


def _sc_gather(table, idx):
    mesh = plsc.VectorSubcoreMesh(core_axis_name="c", subcore_axis_name="s")

    @functools.partial(
        pl.kernel,
        mesh=mesh,
        out_type=jax.ShapeDtypeStruct((M_LANE, H1P // 2), jnp.int32),
        scratch_types=[
            pltpu.VMEM((_PER_W,), jnp.int32),
            pltpu.VMEM((_GCHUNK, H1P // 2), jnp.int32),
            pltpu.VMEM((_GCHUNK, H1P // 2), jnp.int32),
            pltpu.SemaphoreType.DMA,
            pltpu.SemaphoreType.DMA,
        ],
    )
    def gather_kernel(table_hbm, idx_hbm, out_hbm, idx_v, rows_v0, rows_v1,
                      sem0, sem1):
        wid = lax.axis_index("s") * _NC + lax.axis_index("c")
        base = wid * _PER_W
        pltpu.sync_copy(idx_hbm.at[pl.ds(base, _PER_W)], idx_v)
        bufs = (rows_v0, rows_v1)
        sems = (sem0, sem1)
        cps = {}
        cps[0] = pltpu.async_copy(
            table_hbm.at[idx_v.at[pl.ds(0, _GCHUNK)]], bufs[0], sems[0])
        for c in range(_NCHUNK):
            if c + 1 < _NCHUNK:
                cps[c + 1] = pltpu.async_copy(
                    table_hbm.at[idx_v.at[pl.ds((c + 1) * _GCHUNK, _GCHUNK)]],
                    bufs[(c + 1) % 2], sems[(c + 1) % 2])
            cps[c].wait()
            pltpu.sync_copy(bufs[c % 2],
                            out_hbm.at[pl.ds(base + c * _GCHUNK, _GCHUNK)])

    return gather_kernel(table, idx)


def _k3_body(g_ref, lane_ref, w1l_ref, w2_ref, b2_ref, w3_ref, b3_ref,
             w4_ref, b4_ref, out_ref):
    lo, hi = _unpack_bf16_pair(g_ref[...])
    x1 = (jnp.concatenate([lo, hi], axis=1)
          + jnp.dot(lane_ref[...].astype(BF), w1l_ref[...],
                    preferred_element_type=jnp.float32))
    h1 = jnp.maximum(x1, 0.0).astype(BF)
    h2 = jnp.maximum(
        jnp.dot(h1, w2_ref[...], preferred_element_type=jnp.float32)
        + b2_ref[...], 0.0).astype(BF)
    h3 = jnp.maximum(
        jnp.dot(h2, w3_ref[...], preferred_element_type=jnp.float32)
        + b3_ref[...], 0.0).astype(BF)
    out_ref[...] = (jnp.dot(h3, w4_ref[...], preferred_element_type=jnp.float32)
                    + b4_ref[...])


def _mlp_tail(gathered, lane, w1l, w2p, b2p, w3p, b3r, w4, b4r):
    grid = M_LANE // LBLK
    return pl.pallas_call(
        _k3_body,
        grid=(grid,),
        in_specs=[
            pl.BlockSpec((LBLK, H1P // 2), lambda i: (i, 0)),
            pl.BlockSpec((LBLK, D_LANE), lambda i: (i, 0)),
            pl.BlockSpec((D_LANE, H1P), lambda i: (0, 0)),
            pl.BlockSpec((H1P, H2P), lambda i: (0, 0)),
            pl.BlockSpec((1, H2P), lambda i: (0, 0)),
            pl.BlockSpec((H2P, H3), lambda i: (0, 0)),
            pl.BlockSpec((1, H3), lambda i: (0, 0)),
            pl.BlockSpec((H3, 1), lambda i: (0, 0)),
            pl.BlockSpec((1, 1), lambda i: (0, 0)),
        ],
        out_specs=pl.BlockSpec((LBLK, 1), lambda i: (i, 0)),
        out_shape=jax.ShapeDtypeStruct((M_LANE, 1), jnp.float32),
    )(gathered, lane, w1l, w2p, b2p, w3p, b3r, w4, b4r)


def _k4_body(s_ref, seg_ref, out_ref, m_sc, den_sc):
    m_sc[...] = jnp.full((N_OBS, 1), NEG, jnp.float32)
    den_sc[...] = jnp.zeros((N_OBS, 1), jnp.float32)

    for r in range(NROW):
        s_row = s_ref[r]
        seg_row = seg_ref[r]
        for c in range(N_OBS // SCHUNK):
            ids = c * SCHUNK + lax.broadcasted_iota(jnp.int32, (SCHUNK, SROW), 0)
            oh = ids == seg_row
            val = jnp.where(oh, jnp.broadcast_to(s_row, (SCHUNK, SROW)), NEG)
            mx = jnp.max(val, axis=1)[:, None]
            sl = pl.ds(c * SCHUNK, SCHUNK)
            m_sc[sl, :] = jnp.maximum(m_sc[sl, :], mx)

    for r in range(NROW):
        s_row = s_ref[r]
        seg_row = seg_ref[r]
        m_g = jnp.full((1, SROW), NEG, jnp.float32)
        for c in range(N_OBS // SCHUNK):
            ids = c * SCHUNK + lax.broadcasted_iota(jnp.int32, (SCHUNK, SROW), 0)
            oh = ids == seg_row
            m_c = m_sc[pl.ds(c * SCHUNK, SCHUNK), :]
            g = jnp.where(oh, jnp.broadcast_to(m_c, (SCHUNK, SROW)), NEG)
            m_g = jnp.maximum(m_g, jnp.max(g, axis=0)[None, :])
        e = jnp.exp(s_row - m_g)
        out_ref[r] = e
        for c in range(N_OBS // SCHUNK):
            ids = c * SCHUNK + lax.broadcasted_iota(jnp.int32, (SCHUNK, SROW), 0)
            oh = ids == seg_row
            part = jnp.sum(
                jnp.where(oh, jnp.broadcast_to(e, (SCHUNK, SROW)), 0.0),
                axis=1)[:, None]
            sl = pl.ds(c * SCHUNK, SCHUNK)
            den_sc[sl, :] = den_sc[sl, :] + part

    for r in range(NROW):
        seg_row = seg_ref[r]
        d_g = jnp.zeros((1, SROW), jnp.float32)
        for c in range(N_OBS // SCHUNK):
            ids = c * SCHUNK + lax.broadcasted_iota(jnp.int32, (SCHUNK, SROW), 0)
            oh = ids == seg_row
            d_c = den_sc[pl.ds(c * SCHUNK, SCHUNK), :]
            g = jnp.where(oh, jnp.broadcast_to(d_c, (SCHUNK, SROW)), 0.0)
            d_g = jnp.maximum(d_g, jnp.max(g, axis=0)[None, :])
        out_ref[r] = out_ref[r] / d_g


def _segment_softmax(scores, seg):
    s3 = scores.reshape(NROW, 1, SROW)
    seg3 = seg.reshape(NROW, 1, SROW)
    full3 = pl.BlockSpec((NROW, 1, SROW), lambda: (0, 0, 0))

    out3 = pl.pallas_call(
        _k4_body,
        in_specs=[full3, full3],
        out_specs=full3,
        out_shape=jax.ShapeDtypeStruct((NROW, 1, SROW), jnp.float32),
        scratch_shapes=[
            pltpu.VMEM((N_OBS, 1), jnp.float32),
            pltpu.VMEM((N_OBS, 1), jnp.float32),
        ],
    )(s3, seg3)

    return out3.reshape(M_LANE, 1)


def kernel(obs_encoding, lane_encoding, aggregated_info, same_obs_mask,
           W1, b1, W2, b2, W3, b3, W4, b4):
    seg = same_obs_mask[:, 0].astype(jnp.int32)

    w1o = jnp.pad(W1[:D_OBS], ((0, 0), (0, H1P - 600))).astype(BF)
    w1a = jnp.pad(W1[D_OBS:D_OBS + D_AGG], ((0, 0), (0, H1P - 600))).astype(BF)
    w1l = jnp.pad(W1[D_OBS + D_AGG:], ((0, 0), (0, H1P - 600))).astype(BF)
    b1p = jnp.pad(b1, (0, H1P - 600))[None, :]
    w2p = jnp.pad(W2, ((0, H1P - 600), (0, H2P - 100))).astype(BF)
    b2p = jnp.pad(b2, (0, H2P - 100))[None, :]
    w3p = jnp.pad(W3, ((0, H2P - 100), (0, 0))).astype(BF)
    b3r = b3[None, :]
    b4r = b4[None, :]

    h0 = _obstacle_head(obs_encoding, aggregated_info, w1o, w1a, b1p)
    gathered = _sc_gather(h0, seg)
    scores = _mlp_tail(gathered, lane_encoding, w1l, w2p, b2p,
                       w3p, b3r, W4, b4r)
    return _segment_softmax(scores, seg)


# speedup vs baseline: 4.5482x; 1.2647x over previous
import functools

import jax
import jax.numpy as jnp
from jax import lax
from jax.experimental import pallas as pl
from jax.experimental.pallas import tpu as pltpu
from jax.experimental.pallas import tpu_sc as plsc

N_OBS = 2048
M_LANE = 16384
D_OBS = 128
D_LANE = 128
D_AGG = 1024
H1P = 768
H2P = 128
H3 = 16

LBLK = 2048
SROW = 2048
NROW = M_LANE // SROW
SCHUNK = 512
NEG = -1e30

BF = jnp.bfloat16


def _pack_bf16_pair(a, b):
    ba = jax.lax.bitcast_convert_type(a, jnp.int32)
    bb = jax.lax.bitcast_convert_type(b, jnp.int32)
    ba = ba + 32767 + ((ba >> 16) & 1)
    bb = bb + 32767 + ((bb >> 16) & 1)
    return (bb & -65536) | ((ba >> 16) & 65535)


def _unpack_bf16_pair(p):
    a = jax.lax.bitcast_convert_type(p << 16, jnp.float32)
    b = jax.lax.bitcast_convert_type(p & -65536, jnp.float32)
    return a, b


def _k1_body(obs_ref, agg_ref, w1o_ref, w1a_ref, b1_ref, out_ref):
    acc = (
        jnp.dot(obs_ref[...].astype(BF), w1o_ref[...],
                preferred_element_type=jnp.float32)
        + jnp.dot(agg_ref[...].astype(BF), w1a_ref[...],
                  preferred_element_type=jnp.float32)
        + b1_ref[...]
    )
    out_ref[...] = _pack_bf16_pair(acc[:, :H1P // 2], acc[:, H1P // 2:])


def _obstacle_head(obs, agg, w1o, w1a, b1p):
    rb = 512
    grid = N_OBS // rb
    return pl.pallas_call(
        _k1_body,
        grid=(grid,),
        in_specs=[
            pl.BlockSpec((rb, D_OBS), lambda i: (i, 0)),
            pl.BlockSpec((rb, D_AGG), lambda i: (i, 0)),
            pl.BlockSpec((D_OBS, H1P), lambda i: (0, 0)),
            pl.BlockSpec((D_AGG, H1P), lambda i: (0, 0)),
            pl.BlockSpec((1, H1P), lambda i: (0, 0)),
        ],
        out_specs=pl.BlockSpec((rb, H1P // 2), lambda i: (i, 0)),
        out_shape=jax.ShapeDtypeStruct((N_OBS, H1P // 2), jnp.int32),
    )(obs, agg, w1o, w1a, b1p)


_NC = 2
_NS = 16
_NW = _NC * _NS
_PER_W = M_LANE // _NW
_GCHUNK = 128
_NCHUNK = _PER_W // _GCHUNK


def _sc_gather(table, idx):
    mesh = plsc.VectorSubcoreMesh(core_axis_name="c", subcore_axis_name="s")

    @functools.partial(
        pl.kernel,
        mesh=mesh,
        out_type=jax.ShapeDtypeStruct((M_LANE, H1P // 2), jnp.int32),
        scratch_types=[
            pltpu.VMEM((_PER_W,), jnp.int32),
            pltpu.VMEM((_GCHUNK, H1P // 2), jnp.int32),
            pltpu.VMEM((_GCHUNK, H1P // 2), jnp.int32),
            pltpu.SemaphoreType.DMA,
            pltpu.SemaphoreType.DMA,
        ],
    )
    def gather_kernel(table_hbm, idx_hbm, out_hbm, idx_v, rows_v0, rows_v1,
                      sem0, sem1):
        wid = lax.axis_index("s") * _NC + lax.axis_index("c")
        base = wid * _PER_W
        pltpu.sync_copy(idx_hbm.at[pl.ds(base, _PER_W)], idx_v)
        bufs = (rows_v0, rows_v1)
        sems = (sem0, sem1)
        cps = {}
        cps[0] = pltpu.async_copy(
            table_hbm.at[idx_v.at[pl.ds(0, _GCHUNK)]], bufs[0], sems[0])
        for c in range(_NCHUNK):
            if c + 1 < _NCHUNK:
                cps[c + 1] = pltpu.async_copy(
                    table_hbm.at[idx_v.at[pl.ds((c + 1) * _GCHUNK, _GCHUNK)]],
                    bufs[(c + 1) % 2], sems[(c + 1) % 2])
            cps[c].wait()
            pltpu.sync_copy(bufs[c % 2],
                            out_hbm.at[pl.ds(base + c * _GCHUNK, _GCHUNK)])

    return gather_kernel(table, idx)


def _k3_body(g_ref, lane_ref, w1l_ref, w2_ref, b2_ref, w3_ref, b3_ref,
             w4_ref, b4_ref, out_ref, gmax_ref):
    lo, hi = _unpack_bf16_pair(g_ref[...])
    x1 = (jnp.concatenate([lo, hi], axis=1)
          + jnp.dot(lane_ref[...].astype(BF), w1l_ref[...],
                    preferred_element_type=jnp.float32))
    h1 = jnp.maximum(x1, 0.0).astype(BF)
    h2 = jnp.maximum(
        jnp.dot(h1, w2_ref[...], preferred_element_type=jnp.float32)
        + b2_ref[...], 0.0).astype(BF)
    h3 = jnp.maximum(
        jnp.dot(h2, w3_ref[...], preferred_element_type=jnp.float32)
        + b3_ref[...], 0.0).astype(BF)
    scores = (jnp.dot(h3, w4_ref[...], preferred_element_type=jnp.float32)
              + b4_ref[...])
    out_ref[...] = scores
    blk_max = jnp.max(scores)[None, None]

    @pl.when(pl.program_id(0) == 0)
    def _init():
        gmax_ref[...] = blk_max

    gmax_ref[...] = jnp.maximum(gmax_ref[...], blk_max)


def _mlp_tail(gathered, lane, w1l, w2p, b2p, w3p, b3r, w4, b4r):
    grid = M_LANE // LBLK
    return pl.pallas_call(
        _k3_body,
        grid=(grid,),
        in_specs=[
            pl.BlockSpec((LBLK, H1P // 2), lambda i: (i, 0)),
            pl.BlockSpec((LBLK, D_LANE), lambda i: (i, 0)),
            pl.BlockSpec((D_LANE, H1P), lambda i: (0, 0)),
            pl.BlockSpec((H1P, H2P), lambda i: (0, 0)),
            pl.BlockSpec((1, H2P), lambda i: (0, 0)),
            pl.BlockSpec((H2P, H3), lambda i: (0, 0)),
            pl.BlockSpec((1, H3), lambda i: (0, 0)),
            pl.BlockSpec((H3, 1), lambda i: (0, 0)),
            pl.BlockSpec((1, 1), lambda i: (0, 0)),
        ],
        out_specs=[pl.BlockSpec((LBLK, 1), lambda i: (i, 0)),
                   pl.BlockSpec((1, 1), lambda i: (0, 0))],
        out_shape=[jax.ShapeDtypeStruct((M_LANE, 1), jnp.float32),
                   jax.ShapeDtypeStruct((1, 1), jnp.float32)],
    )(gathered, lane, w1l, w2p, b2p, w3p, b3r, w4, b4r)


def _k4_body(s_ref, seg_ref, gmax_ref, out_ref, den_sc):
    den_sc[...] = jnp.zeros((N_OBS, 1), jnp.float32)

    for r in range(NROW):
        seg_row = seg_ref[r]
        e = jnp.exp(s_ref[r] - gmax_ref[...])
        out_ref[r] = e
        for c in range(N_OBS // SCHUNK):
            ids = c * SCHUNK + lax.broadcasted_iota(jnp.int32, (SCHUNK, SROW), 0)
            oh = ids == seg_row
            part = jnp.sum(
                jnp.where(oh, jnp.broadcast_to(e, (SCHUNK, SROW)), 0.0),
                axis=1)[:, None]
            sl = pl.ds(c * SCHUNK, SCHUNK)
            den_sc[sl, :] = den_sc[sl, :] + part

    for r in range(NROW):
        seg_row = seg_ref[r]
        d_g = jnp.zeros((1, SROW), jnp.float32)
        for c in range(N_OBS // SCHUNK):
            ids = c * SCHUNK + lax.broadcasted_iota(jnp.int32, (SCHUNK, SROW), 0)
            oh = ids == seg_row
            d_c = den_sc[pl.ds(c * SCHUNK, SCHUNK), :]
            g = jnp.where(oh, jnp.broadcast_to(d_c, (SCHUNK, SROW)), 0.0)
            d_g = jnp.maximum(d_g, jnp.max(g, axis=0)[None, :])
        out_ref[r] = out_ref[r] / d_g


def _segment_softmax(scores, gmax, seg):
    s3 = scores.reshape(NROW, 1, SROW)
    seg3 = seg.reshape(NROW, 1, SROW)
    full3 = pl.BlockSpec((NROW, 1, SROW), lambda: (0, 0, 0))

    out3 = pl.pallas_call(
        _k4_body,
        in_specs=[full3, full3, pl.BlockSpec((1, 1), lambda: (0, 0))],
        out_specs=full3,
        out_shape=jax.ShapeDtypeStruct((NROW, 1, SROW), jnp.float32),
        scratch_shapes=[
            pltpu.VMEM((N_OBS, 1), jnp.float32),
        ],
    )(s3, seg3, gmax)

    return out3.reshape(M_LANE, 1)


def kernel(obs_encoding, lane_encoding, aggregated_info, same_obs_mask,
           W1, b1, W2, b2, W3, b3, W4, b4):
    seg = same_obs_mask[:, 0].astype(jnp.int32)

    w1o = jnp.pad(W1[:D_OBS], ((0, 0), (0, H1P - 600))).astype(BF)
    w1a = jnp.pad(W1[D_OBS:D_OBS + D_AGG], ((0, 0), (0, H1P - 600))).astype(BF)
    w1l = jnp.pad(W1[D_OBS + D_AGG:], ((0, 0), (0, H1P - 600))).astype(BF)
    b1p = jnp.pad(b1, (0, H1P - 600))[None, :]
    w2p = jnp.pad(W2, ((0, H1P - 600), (0, H2P - 100))).astype(BF)
    b2p = jnp.pad(b2, (0, H2P - 100))[None, :]
    w3p = jnp.pad(W3, ((0, H2P - 100), (0, 0))).astype(BF)
    b3r = b3[None, :]
    b4r = b4[None, :]

    h0 = _obstacle_head(obs_encoding, aggregated_info, w1o, w1a, b1p)
    gathered = _sc_gather(h0, seg)
    scores, gmax = _mlp_tail(gathered, lane_encoding, w1l, w2p, b2p,
                             w3p, b3r, W4, b4r)
    return _segment_softmax(scores, gmax, seg)


# speedup vs baseline: 5.5214x; 1.2140x over previous
import functools

import jax
import jax.numpy as jnp
from jax import lax
from jax.experimental import pallas as pl
from jax.experimental.pallas import tpu as pltpu
from jax.experimental.pallas import tpu_sc as plsc

N_OBS = 2048
M_LANE = 16384
D_OBS = 128
D_LANE = 128
D_AGG = 1024
H1P = 768
H2P = 128
H3 = 16

LBLK = 2048
SROW = 2048
NROW = M_LANE // SROW
SCHUNK = 512
NEG = -1e30

BF = jnp.bfloat16


def _pack_bf16_pair(a, b):
    ba = jax.lax.bitcast_convert_type(a, jnp.int32)
    bb = jax.lax.bitcast_convert_type(b, jnp.int32)
    ba = ba + 32767 + ((ba >> 16) & 1)
    bb = bb + 32767 + ((bb >> 16) & 1)
    return (bb & -65536) | ((ba >> 16) & 65535)


def _unpack_bf16_pair(p):
    a = jax.lax.bitcast_convert_type(p << 16, jnp.float32)
    b = jax.lax.bitcast_convert_type(p & -65536, jnp.float32)
    return a, b


def _k1_body(obs_ref, agg_ref, w1o_ref, w1a_ref, b1_ref, out_ref):
    acc = (
        jnp.dot(obs_ref[...].astype(BF), w1o_ref[...],
                preferred_element_type=jnp.float32)
        + jnp.dot(agg_ref[...].astype(BF), w1a_ref[...],
                  preferred_element_type=jnp.float32)
        + b1_ref[...]
    )
    out_ref[...] = _pack_bf16_pair(acc[:, :H1P // 2], acc[:, H1P // 2:])


def _obstacle_head(obs, agg, w1o, w1a, b1p):
    rb = 512
    grid = N_OBS // rb
    return pl.pallas_call(
        _k1_body,
        grid=(grid,),
        in_specs=[
            pl.BlockSpec((rb, D_OBS), lambda i: (i, 0)),
            pl.BlockSpec((rb, D_AGG), lambda i: (i, 0)),
            pl.BlockSpec((D_OBS, H1P), lambda i: (0, 0)),
            pl.BlockSpec((D_AGG, H1P), lambda i: (0, 0)),
            pl.BlockSpec((1, H1P), lambda i: (0, 0)),
        ],
        out_specs=pl.BlockSpec((rb, H1P // 2), lambda i: (i, 0)),
        out_shape=jax.ShapeDtypeStruct((N_OBS, H1P // 2), jnp.int32),
    )(obs, agg, w1o, w1a, b1p)


_NC = 2
_NS = 16
_NW = _NC * _NS
_PER_W = M_LANE // _NW
_GCHUNK = 128
_NCHUNK = _PER_W // _GCHUNK


def _sc_gather(table, idx):
    mesh = plsc.VectorSubcoreMesh(core_axis_name="c", subcore_axis_name="s")

    @functools.partial(
        pl.kernel,
        mesh=mesh,
        out_type=jax.ShapeDtypeStruct((M_LANE, H1P // 2), jnp.int32),
        scratch_types=[
            pltpu.VMEM((_PER_W,), jnp.int32),
            pltpu.VMEM((_GCHUNK, H1P // 2), jnp.int32),
            pltpu.VMEM((_GCHUNK, H1P // 2), jnp.int32),
            pltpu.SemaphoreType.DMA,
            pltpu.SemaphoreType.DMA,
        ],
    )
    def gather_kernel(table_hbm, idx_hbm, out_hbm, idx_v, rows_v0, rows_v1,
                      sem0, sem1):
        wid = lax.axis_index("s") * _NC + lax.axis_index("c")
        base = wid * _PER_W
        pltpu.sync_copy(idx_hbm.at[pl.ds(base, _PER_W)], idx_v)
        bufs = (rows_v0, rows_v1)
        sems = (sem0, sem1)
        cps = {}
        cps[0] = pltpu.async_copy(
            table_hbm.at[idx_v.at[pl.ds(0, _GCHUNK)]], bufs[0], sems[0])
        for c in range(_NCHUNK):
            if c + 1 < _NCHUNK:
                cps[c + 1] = pltpu.async_copy(
                    table_hbm.at[idx_v.at[pl.ds((c + 1) * _GCHUNK, _GCHUNK)]],
                    bufs[(c + 1) % 2], sems[(c + 1) % 2])
            cps[c].wait()
            pltpu.sync_copy(bufs[c % 2],
                            out_hbm.at[pl.ds(base + c * _GCHUNK, _GCHUNK)])

    return gather_kernel(table, idx)


def _k3_body(g_ref, lane_ref, w1l_ref, w2_ref, b2_ref, w3_ref, b3_ref,
             w4_ref, b4_ref, out_ref, gmax_ref):
    lo, hi = _unpack_bf16_pair(g_ref[...])
    x1 = (jnp.concatenate([lo, hi], axis=1)
          + jnp.dot(lane_ref[...].astype(BF), w1l_ref[...],
                    preferred_element_type=jnp.float32))
    h1 = jnp.maximum(x1, 0.0).astype(BF)
    h2 = jnp.maximum(
        jnp.dot(h1, w2_ref[...], preferred_element_type=jnp.float32)
        + b2_ref[...], 0.0).astype(BF)
    h3 = jnp.maximum(
        jnp.dot(h2, w3_ref[...], preferred_element_type=jnp.float32)
        + b3_ref[...], 0.0).astype(BF)
    scores = (jnp.dot(h3, w4_ref[...], preferred_element_type=jnp.float32)
              + b4_ref[...])
    out_ref[...] = scores
    blk_max = jnp.max(scores)[None, None]

    @pl.when(pl.program_id(0) == 0)
    def _init():
        gmax_ref[...] = blk_max

    gmax_ref[...] = jnp.maximum(gmax_ref[...], blk_max)


def _mlp_tail(gathered, lane, w1l, w2p, b2p, w3p, b3r, w4, b4r):
    grid = M_LANE // LBLK
    return pl.pallas_call(
        _k3_body,
        grid=(grid,),
        in_specs=[
            pl.BlockSpec((LBLK, H1P // 2), lambda i: (i, 0)),
            pl.BlockSpec((LBLK, D_LANE), lambda i: (i, 0)),
            pl.BlockSpec((D_LANE, H1P), lambda i: (0, 0)),
            pl.BlockSpec((H1P, H2P), lambda i: (0, 0)),
            pl.BlockSpec((1, H2P), lambda i: (0, 0)),
            pl.BlockSpec((H2P, H3), lambda i: (0, 0)),
            pl.BlockSpec((1, H3), lambda i: (0, 0)),
            pl.BlockSpec((H3, 1), lambda i: (0, 0)),
            pl.BlockSpec((1, 1), lambda i: (0, 0)),
        ],
        out_specs=[pl.BlockSpec((LBLK, 1), lambda i: (i, 0)),
                   pl.BlockSpec((1, 1), lambda i: (0, 0))],
        out_shape=[jax.ShapeDtypeStruct((M_LANE, 1), jnp.float32),
                   jax.ShapeDtypeStruct((1, 1), jnp.float32)],
    )(gathered, lane, w1l, w2p, b2p, w3p, b3r, w4, b4r)


_SM_T = 16
_SM_J = 8
_SM_B = 128
_SM_LPT = _SM_J * _SM_B


def _sc_softmax(scores, gmax, seg):
    s3 = scores.reshape(_SM_T, _SM_J, _SM_B)
    seg3 = seg.reshape(_SM_T, _SM_J, _SM_B)
    g16 = jnp.broadcast_to(gmax.reshape(()), (16,))
    zer = jnp.zeros((N_OBS,), jnp.float32)
    mesh = plsc.VectorSubcoreMesh(core_axis_name="c", subcore_axis_name="s")

    @functools.partial(
        pl.kernel,
        mesh=mesh,
        out_type=jax.ShapeDtypeStruct((_SM_T, _SM_J, _SM_B), jnp.float32),
        scratch_types=[
            pltpu.VMEM((_SM_J, _SM_B), jnp.float32),
            pltpu.VMEM((_SM_J, _SM_B), jnp.int32),
            pltpu.VMEM((N_OBS,), jnp.float32),
            pltpu.VMEM((16,), jnp.float32),
            pltpu.VMEM_SHARED((N_OBS,), jnp.float32),
        ],
        compiler_params=pltpu.CompilerParams(needs_layout_passes=False),
    )
    def softmax_kernel(s_hbm, seg_hbm, g_hbm, z_hbm, out_hbm,
                       s_v, seg_v, den_v, g_v, den_sp):
        t = lax.axis_index("s")
        core = lax.axis_index("c")
        on = core == 0

        @pl.when(on & (t == 0))
        def _init():
            pltpu.sync_copy(z_hbm, den_sp)

        @pl.when(on)
        def _exp():
            pltpu.sync_copy(s_hbm.at[t], s_v)
            pltpu.sync_copy(seg_hbm.at[t], seg_v)
            pltpu.sync_copy(g_hbm, g_v)
            g = g_v[...]
            for j in range(_SM_J):
                for u in range(_SM_B // 16):
                    sl = pl.ds(u * 16, 16)
                    s_v[j, sl] = jnp.exp(s_v[j, sl] - g)

        plsc.subcore_barrier()

        @pl.when(on)
        def _scatter_add():
            for j in range(_SM_J):
                pltpu.sync_copy(s_v.at[j], den_sp.at[seg_v.at[j]], add=True)

        plsc.subcore_barrier()

        @pl.when(on)
        def _gather_div():
            pltpu.sync_copy(den_sp, den_v)
            for j in range(_SM_J):
                for u in range(_SM_B // 16):
                    sl = pl.ds(u * 16, 16)
                    idx = seg_v[j, sl]
                    dg = plsc.load_gather(den_v, [idx])
                    s_v[j, sl] = s_v[j, sl] / dg
            pltpu.sync_copy(s_v, out_hbm.at[t])

    return softmax_kernel(s3, seg3, g16, zer).reshape(M_LANE, 1)


def kernel(obs_encoding, lane_encoding, aggregated_info, same_obs_mask,
           W1, b1, W2, b2, W3, b3, W4, b4):
    seg = same_obs_mask[:, 0].astype(jnp.int32)

    w1o = jnp.pad(W1[:D_OBS], ((0, 0), (0, H1P - 600))).astype(BF)
    w1a = jnp.pad(W1[D_OBS:D_OBS + D_AGG], ((0, 0), (0, H1P - 600))).astype(BF)
    w1l = jnp.pad(W1[D_OBS + D_AGG:], ((0, 0), (0, H1P - 600))).astype(BF)
    b1p = jnp.pad(b1, (0, H1P - 600))[None, :]
    w2p = jnp.pad(W2, ((0, H1P - 600), (0, H2P - 100))).astype(BF)
    b2p = jnp.pad(b2, (0, H2P - 100))[None, :]
    w3p = jnp.pad(W3, ((0, H2P - 100), (0, 0))).astype(BF)
    b3r = b3[None, :]
    b4r = b4[None, :]

    h0 = _obstacle_head(obs_encoding, aggregated_info, w1o, w1a, b1p)
    gathered = _sc_gather(h0, seg)
    scores, gmax = _mlp_tail(gathered, lane_encoding, w1l, w2p, b2p,
                             w3p, b3r, W4, b4r)
    return _sc_softmax(scores, gmax, seg)
